# Initial kernel scaffold; baseline (speedup 1.0000x reference)
import jax
import jax.numpy as jnp
from jax.experimental import pallas as pl


def kernel(x, edge_index, W1, b1, W2, b2, Wl, bl):
    raise NotImplementedError("write your pallas kernel here")


# speedup vs baseline: 5.1648x; 5.1648x over previous
import functools

import jax
import jax.numpy as jnp
from jax import lax
from jax.experimental import pallas as pl
from jax.experimental.pallas import tpu as pltpu
from jax.experimental.pallas import tpu_sc as plsc

NC = 2
NS = 16
LANES = 16
CHUNK = 128


def _agg_body(n_nodes, n_chunks, width, g_hbm, gh_hbm, src_hbm, dst_hbm,
              out_hbm, idx_s, idx_d, rows_v, acc_sh, sem):
  c = lax.axis_index("c")
  s = lax.axis_index("s")
  w = c * NS + s
  rows_pt = n_nodes // NS

  pltpu.sync_copy(gh_hbm.at[pl.ds(s * rows_pt, rows_pt)],
                  acc_sh.at[pl.ds(s * rows_pt, rows_pt)])
  plsc.subcore_barrier()

  def step(j, carry):
    pltpu.sync_copy(src_hbm.at[w, j], idx_s)
    pltpu.sync_copy(dst_hbm.at[w, j], idx_d)
    pltpu.async_copy(g_hbm.at[idx_s], rows_v, sem).wait()
    pltpu.sync_copy(rows_v, acc_sh.at[idx_d], add=True)
    return carry

  lax.fori_loop(0, n_chunks, step, 0)
  plsc.subcore_barrier()
  pltpu.sync_copy(acc_sh.at[pl.ds(s * rows_pt, rows_pt)],
                  out_hbm.at[c, pl.ds(s * rows_pt, rows_pt)])


def _sc_aggregate(g, gh, src_resh, dst_resh):
  n_nodes, width = g.shape
  n_chunks = src_resh.shape[1]
  mesh = plsc.VectorSubcoreMesh(core_axis_name="c", subcore_axis_name="s")
  return pl.kernel(
      functools.partial(_agg_body, n_nodes, n_chunks, width),
      out_type=jax.ShapeDtypeStruct((NC, n_nodes, width), jnp.float32),
      mesh=mesh,
      scratch_types=[
          pltpu.VMEM((CHUNK,), jnp.int32),
          pltpu.VMEM((CHUNK,), jnp.int32),
          pltpu.VMEM((CHUNK, width), jnp.float32),
          pltpu.VMEM_SHARED((n_nodes, width), jnp.float32),
          pltpu.SemaphoreType.DMA,
      ],
  )(g, gh, src_resh, dst_resh)


def _pre_body(degp_ref, x_ref, w1t_ref, dinv_ref, g_ref, gh_ref):
  deg = degp_ref[0] + degp_ref[1]
  deg = deg[:, 0:1] + 1.0
  dinv = lax.rsqrt(deg)
  h = jnp.dot(x_ref[...], w1t_ref[...], preferred_element_type=jnp.float32)
  g = h * dinv
  dinv_ref[...] = dinv
  g_ref[...] = g
  gh_ref[...] = g * 0.5


def _tc_pre(degp, x, w1t):
  n, d = x.shape
  h = w1t.shape[1]
  return pl.pallas_call(
      _pre_body,
      out_shape=[
          jax.ShapeDtypeStruct((n, 1), jnp.float32),
          jax.ShapeDtypeStruct((n, h), jnp.float32),
          jax.ShapeDtypeStruct((n, h), jnp.float32),
      ],
  )(degp, x, w1t)


def _mid_body(p_ref, dinv_ref, b1_ref, w2t_ref, g_ref, gh_ref):
  h = p_ref[0] + p_ref[1]
  h = h * dinv_ref[...] + b1_ref[...]
  h = jnp.maximum(h, 0.0)
  g = jnp.dot(h, w2t_ref[...], preferred_element_type=jnp.float32)
  g = g * dinv_ref[...]
  g_ref[...] = g
  gh_ref[...] = g * 0.5


def _tc_mid(p, dinv, b1, w2t):
  n = p.shape[1]
  h = w2t.shape[1]
  return pl.pallas_call(
      _mid_body,
      out_shape=[
          jax.ShapeDtypeStruct((n, h), jnp.float32),
          jax.ShapeDtypeStruct((n, h), jnp.float32),
      ],
  )(p, dinv, b1, w2t)


def _post_body(p_ref, dinv_ref, b2_ref, wlt_ref, bl_ref, out_ref):
  h = p_ref[0] + p_ref[1]
  h = h * dinv_ref[...] + b2_ref[...]
  h = jnp.maximum(h, 0.0)
  out_ref[...] = (
      jnp.dot(h, wlt_ref[...], preferred_element_type=jnp.float32)
      + bl_ref[...])


def _tc_post(p, dinv, b2, wlt, bl):
  n = p.shape[1]
  n_out = wlt.shape[1]
  return pl.pallas_call(
      _post_body,
      out_shape=jax.ShapeDtypeStruct((n, n_out), jnp.float32),
  )(p, dinv, b2, wlt, bl)


def kernel(x, edge_index, W1, b1, W2, b2, Wl, bl):
  n_nodes, d = x.shape
  n_edges = edge_index.shape[1]
  n_workers = NC * NS

  n_pad = NS * 8 * ((n_nodes + NS * 8 - 1) // (NS * 8))
  x = jnp.pad(x, ((0, n_pad - n_nodes), (0, 0)))

  per_w = CHUNK * 8 * (
      (n_edges + n_workers * CHUNK * 8 - 1) // (n_workers * CHUNK * 8))
  e_pad = n_workers * per_w
  edge_index = jnp.pad(edge_index, ((0, 0), (0, e_pad - n_edges)),
                       constant_values=n_nodes)

  src32 = edge_index[0].reshape(n_workers, per_w // CHUNK, CHUNK)
  dst32 = edge_index[1].reshape(n_workers, per_w // CHUNK, CHUNK)

  ones_w = jnp.ones((n_pad, d), jnp.float32)
  zeros_w = jnp.zeros((n_pad, d), jnp.float32)
  degp = _sc_aggregate(ones_w, zeros_w, dst32, dst32)
  dinv, g1, gh1 = _tc_pre(degp, x, W1.T)
  p1 = _sc_aggregate(g1, gh1, src32, dst32)
  g2, gh2 = _tc_mid(p1, dinv, b1.reshape(1, -1), W2.T)
  p2 = _sc_aggregate(g2, gh2, src32, dst32)
  out = _tc_post(p2, dinv, b2.reshape(1, -1), Wl.T, bl.reshape(1, -1))
  return out[:n_nodes]


# speedup vs baseline: 8.4669x; 1.6393x over previous
import functools

import jax
import jax.numpy as jnp
from jax import lax
from jax.experimental import pallas as pl
from jax.experimental.pallas import tpu as pltpu
from jax.experimental.pallas import tpu_sc as plsc

NC = 2
NS = 16
LANES = 16
CHUNK = 128


def _agg_body(n_nodes, n_chunks, width, g_hbm, gh_hbm, src_hbm, dst_hbm,
              out_hbm, idx_s, idxd0, idxd1, rows0, rows1, acc_sh,
              sg0, sg1, sd0, sd1):
  c = lax.axis_index("c")
  s = lax.axis_index("s")
  w = c * NS + s
  rows_pt = n_nodes // NS

  pltpu.sync_copy(gh_hbm.at[pl.ds(s * rows_pt, rows_pt)],
                  acc_sh.at[pl.ds(s * rows_pt, rows_pt)])
  pltpu.sync_copy(src_hbm.at[w], idx_s)
  plsc.subcore_barrier()

  pltpu.async_copy(dst_hbm.at[w, 0], idxd0, sd0)
  pltpu.async_copy(dst_hbm.at[w, 1], idxd1, sd1)
  pltpu.async_copy(g_hbm.at[idx_s.at[0]], rows0, sg0)
  pltpu.async_copy(g_hbm.at[idx_s.at[1]], rows1, sg1)
  n_half = n_chunks // 2

  def step(k, carry):
    j0 = 2 * k
    pltpu.make_async_copy(g_hbm.at[pl.ds(0, CHUNK)], rows0, sg0).wait()
    pltpu.make_async_copy(dst_hbm.at[w, 0], idxd0, sd0).wait()
    pltpu.sync_copy(rows0, acc_sh.at[idxd0], add=True)

    @pl.when(k < n_half - 1)
    def _():
      pltpu.async_copy(g_hbm.at[idx_s.at[j0 + 2]], rows0, sg0)
      pltpu.async_copy(dst_hbm.at[w, j0 + 2], idxd0, sd0)

    pltpu.make_async_copy(g_hbm.at[pl.ds(0, CHUNK)], rows1, sg1).wait()
    pltpu.make_async_copy(dst_hbm.at[w, 1], idxd1, sd1).wait()
    pltpu.sync_copy(rows1, acc_sh.at[idxd1], add=True)

    @pl.when(k < n_half - 1)
    def _():
      pltpu.async_copy(g_hbm.at[idx_s.at[j0 + 3]], rows1, sg1)
      pltpu.async_copy(dst_hbm.at[w, j0 + 3], idxd1, sd1)

    return carry

  lax.fori_loop(0, n_half, step, 0)
  plsc.subcore_barrier()
  pltpu.sync_copy(acc_sh.at[pl.ds(s * rows_pt, rows_pt)],
                  out_hbm.at[c, pl.ds(s * rows_pt, rows_pt)])


def _sc_aggregate(g, gh, src_resh, dst_resh):
  n_nodes, width = g.shape
  n_chunks = src_resh.shape[1]
  mesh = plsc.VectorSubcoreMesh(core_axis_name="c", subcore_axis_name="s")
  return pl.kernel(
      functools.partial(_agg_body, n_nodes, n_chunks, width),
      out_type=jax.ShapeDtypeStruct((NC, n_nodes, width), jnp.float32),
      mesh=mesh,
      scratch_types=[
          pltpu.VMEM((n_chunks, CHUNK), jnp.int32),
          pltpu.VMEM((CHUNK,), jnp.int32),
          pltpu.VMEM((CHUNK,), jnp.int32),
          pltpu.VMEM((CHUNK, width), jnp.float32),
          pltpu.VMEM((CHUNK, width), jnp.float32),
          pltpu.VMEM_SHARED((n_nodes, width), jnp.float32),
          pltpu.SemaphoreType.DMA,
          pltpu.SemaphoreType.DMA,
          pltpu.SemaphoreType.DMA,
          pltpu.SemaphoreType.DMA,
      ],
  )(g, gh, src_resh, dst_resh)


def _deg_body(n_nodes, n_chunks, width, dst_hbm, ones_hbm, zeros_hbm,
              out_hbm, idx_d, ones_v, acc_sh, sem0, sem1):
  c = lax.axis_index("c")
  s = lax.axis_index("s")
  w = c * NS + s
  rows_pt = n_nodes // NS

  pltpu.sync_copy(zeros_hbm.at[pl.ds(s * rows_pt, rows_pt)],
                  acc_sh.at[pl.ds(s * rows_pt, rows_pt)])
  pltpu.sync_copy(ones_hbm, ones_v)
  pltpu.sync_copy(dst_hbm.at[w], idx_d)
  plsc.subcore_barrier()

  pltpu.async_copy(ones_v, acc_sh.at[idx_d.at[0]], sem0, add=True)
  pltpu.async_copy(ones_v, acc_sh.at[idx_d.at[1]], sem1, add=True)
  n_half = n_chunks // 2

  def step(k, carry):
    j0 = 2 * k
    pltpu.make_async_copy(ones_v, acc_sh.at[idx_d.at[j0]], sem0).wait()

    @pl.when(k < n_half - 1)
    def _():
      pltpu.async_copy(ones_v, acc_sh.at[idx_d.at[j0 + 2]], sem0, add=True)

    pltpu.make_async_copy(ones_v, acc_sh.at[idx_d.at[j0 + 1]], sem1).wait()

    @pl.when(k < n_half - 1)
    def _():
      pltpu.async_copy(ones_v, acc_sh.at[idx_d.at[j0 + 3]], sem1, add=True)

    return carry

  lax.fori_loop(0, n_half, step, 0)
  plsc.subcore_barrier()
  pltpu.sync_copy(acc_sh.at[pl.ds(s * rows_pt, rows_pt)],
                  out_hbm.at[c, pl.ds(s * rows_pt, rows_pt)])


def _sc_degree(dst_resh, ones_blk, zeros_w):
  n_nodes, width = zeros_w.shape
  n_chunks = dst_resh.shape[1]
  mesh = plsc.VectorSubcoreMesh(core_axis_name="c", subcore_axis_name="s")
  return pl.kernel(
      functools.partial(_deg_body, n_nodes, n_chunks, width),
      out_type=jax.ShapeDtypeStruct((NC, n_nodes, width), jnp.float32),
      mesh=mesh,
      scratch_types=[
          pltpu.VMEM((n_chunks, CHUNK), jnp.int32),
          pltpu.VMEM((CHUNK, width), jnp.float32),
          pltpu.VMEM_SHARED((n_nodes, width), jnp.float32),
          pltpu.SemaphoreType.DMA,
          pltpu.SemaphoreType.DMA,
      ],
  )(dst_resh, ones_blk, zeros_w)


def _pre_body(degp_ref, x_ref, w1t_ref, dinv_ref, g_ref, gh_ref):
  deg = degp_ref[0] + degp_ref[1]
  deg = deg[:, 0:1] + 1.0
  dinv = lax.rsqrt(deg)
  h = jnp.dot(x_ref[...], w1t_ref[...], preferred_element_type=jnp.float32)
  g = h * dinv
  dinv_ref[...] = dinv
  g_ref[...] = g
  gh_ref[...] = g * 0.5


def _tc_pre(degp, x, w1t):
  n, d = x.shape
  h = w1t.shape[1]
  return pl.pallas_call(
      _pre_body,
      out_shape=[
          jax.ShapeDtypeStruct((n, 1), jnp.float32),
          jax.ShapeDtypeStruct((n, h), jnp.float32),
          jax.ShapeDtypeStruct((n, h), jnp.float32),
      ],
  )(degp, x, w1t)


def _mid_body(p_ref, dinv_ref, b1_ref, w2t_ref, g_ref, gh_ref):
  h = p_ref[0] + p_ref[1]
  h = h * dinv_ref[...] + b1_ref[...]
  h = jnp.maximum(h, 0.0)
  g = jnp.dot(h, w2t_ref[...], preferred_element_type=jnp.float32)
  g = g * dinv_ref[...]
  g_ref[...] = g
  gh_ref[...] = g * 0.5


def _tc_mid(p, dinv, b1, w2t):
  n = p.shape[1]
  h = w2t.shape[1]
  return pl.pallas_call(
      _mid_body,
      out_shape=[
          jax.ShapeDtypeStruct((n, h), jnp.float32),
          jax.ShapeDtypeStruct((n, h), jnp.float32),
      ],
  )(p, dinv, b1, w2t)


def _post_body(p_ref, dinv_ref, b2_ref, wlt_ref, bl_ref, out_ref):
  h = p_ref[0] + p_ref[1]
  h = h * dinv_ref[...] + b2_ref[...]
  h = jnp.maximum(h, 0.0)
  out_ref[...] = (
      jnp.dot(h, wlt_ref[...], preferred_element_type=jnp.float32)
      + bl_ref[...])


def _tc_post(p, dinv, b2, wlt, bl):
  n = p.shape[1]
  n_out = wlt.shape[1]
  return pl.pallas_call(
      _post_body,
      out_shape=jax.ShapeDtypeStruct((n, n_out), jnp.float32),
  )(p, dinv, b2, wlt, bl)


def kernel(x, edge_index, W1, b1, W2, b2, Wl, bl):
  n_nodes, d = x.shape
  n_edges = edge_index.shape[1]
  n_workers = NC * NS

  n_pad = NS * 8 * ((n_nodes + NS * 8 - 1) // (NS * 8))
  x = jnp.pad(x, ((0, n_pad - n_nodes), (0, 0)))

  per_w = CHUNK * 8 * (
      (n_edges + n_workers * CHUNK * 8 - 1) // (n_workers * CHUNK * 8))
  e_pad = n_workers * per_w
  edge_index = jnp.pad(edge_index, ((0, 0), (0, e_pad - n_edges)),
                       constant_values=n_nodes)

  src32 = edge_index[0].reshape(n_workers, per_w // CHUNK, CHUNK)
  dst32 = edge_index[1].reshape(n_workers, per_w // CHUNK, CHUNK)

  ones_blk = jnp.ones((CHUNK, d), jnp.float32)
  zeros_w = jnp.zeros((n_pad, d), jnp.float32)
  degp = _sc_degree(dst32, ones_blk, zeros_w)
  dinv, g1, gh1 = _tc_pre(degp, x, W1.T)
  p1 = _sc_aggregate(g1, gh1, src32, dst32)
  g2, gh2 = _tc_mid(p1, dinv, b1.reshape(1, -1), W2.T)
  p2 = _sc_aggregate(g2, gh2, src32, dst32)
  out = _tc_post(p2, dinv, b2.reshape(1, -1), Wl.T, bl.reshape(1, -1))
  return out[:n_nodes]


# speedup vs baseline: 27.7616x; 3.2788x over previous
import functools

import jax
import jax.numpy as jnp
from jax import lax
from jax.experimental import pallas as pl
from jax.experimental.pallas import tpu as pltpu
from jax.experimental.pallas import tpu_sc as plsc

NC = 2
NS = 16
LANES = 16
CHUNK = 128


def _agg_body(n_nodes, n_chunks, width, g_hbm, gh_hbm, src_hbm, dst_hbm,
              out_hbm, idx_s, idxd0, idxd1, rows0, rows1, acc_sh,
              sg0, sg1, sd0, sd1):
  c = lax.axis_index("c")
  s = lax.axis_index("s")
  w = c * NS + s
  rows_pt = n_nodes // NS

  pltpu.sync_copy(gh_hbm.at[pl.ds(s * rows_pt, rows_pt)],
                  acc_sh.at[pl.ds(s * rows_pt, rows_pt)])
  pltpu.sync_copy(src_hbm.at[w], idx_s)
  plsc.subcore_barrier()

  pltpu.async_copy(dst_hbm.at[w, 0], idxd0, sd0)
  pltpu.async_copy(dst_hbm.at[w, 1], idxd1, sd1)
  pltpu.async_copy(g_hbm.at[idx_s.at[0]], rows0, sg0)
  pltpu.async_copy(g_hbm.at[idx_s.at[1]], rows1, sg1)
  n_half = n_chunks // 2

  def step(k, carry):
    j0 = 2 * k
    pltpu.make_async_copy(g_hbm.at[pl.ds(0, CHUNK)], rows0, sg0).wait()
    pltpu.make_async_copy(dst_hbm.at[w, 0], idxd0, sd0).wait()
    pltpu.sync_copy(rows0, acc_sh.at[idxd0], add=True)

    @pl.when(k < n_half - 1)
    def _():
      pltpu.async_copy(g_hbm.at[idx_s.at[j0 + 2]], rows0, sg0)
      pltpu.async_copy(dst_hbm.at[w, j0 + 2], idxd0, sd0)

    pltpu.make_async_copy(g_hbm.at[pl.ds(0, CHUNK)], rows1, sg1).wait()
    pltpu.make_async_copy(dst_hbm.at[w, 1], idxd1, sd1).wait()
    pltpu.sync_copy(rows1, acc_sh.at[idxd1], add=True)

    @pl.when(k < n_half - 1)
    def _():
      pltpu.async_copy(g_hbm.at[idx_s.at[j0 + 3]], rows1, sg1)
      pltpu.async_copy(dst_hbm.at[w, j0 + 3], idxd1, sd1)

    return carry

  lax.fori_loop(0, n_half, step, 0)
  plsc.subcore_barrier()
  pltpu.sync_copy(acc_sh.at[pl.ds(s * rows_pt, rows_pt)],
                  out_hbm.at[c, pl.ds(s * rows_pt, rows_pt)])


def _sc_aggregate(g, gh, src_resh, dst_resh):
  n_nodes, width = g.shape
  n_chunks = src_resh.shape[1]
  mesh = plsc.VectorSubcoreMesh(core_axis_name="c", subcore_axis_name="s")
  return pl.kernel(
      functools.partial(_agg_body, n_nodes, n_chunks, width),
      out_type=jax.ShapeDtypeStruct((NC, n_nodes, width), jnp.float32),
      mesh=mesh,
      scratch_types=[
          pltpu.VMEM((n_chunks, CHUNK), jnp.int32),
          pltpu.VMEM((CHUNK,), jnp.int32),
          pltpu.VMEM((CHUNK,), jnp.int32),
          pltpu.VMEM((CHUNK, width), jnp.float32),
          pltpu.VMEM((CHUNK, width), jnp.float32),
          pltpu.VMEM_SHARED((n_nodes, width), jnp.float32),
          pltpu.SemaphoreType.DMA,
          pltpu.SemaphoreType.DMA,
          pltpu.SemaphoreType.DMA,
          pltpu.SemaphoreType.DMA,
      ],
  )(g, gh, src_resh, dst_resh)


def _deg_body(n_nodes, n_chunks, width, dst_hbm, ones_hbm, zeros_hbm,
              out_hbm, idx_d, ones_v, acc_sh, sem0, sem1):
  c = lax.axis_index("c")
  s = lax.axis_index("s")
  w = c * NS + s
  rows_pt = n_nodes // NS

  pltpu.sync_copy(zeros_hbm.at[pl.ds(s * rows_pt, rows_pt)],
                  acc_sh.at[pl.ds(s * rows_pt, rows_pt)])
  pltpu.sync_copy(ones_hbm, ones_v)
  pltpu.sync_copy(dst_hbm.at[w], idx_d)
  plsc.subcore_barrier()

  pltpu.async_copy(ones_v, acc_sh.at[idx_d.at[0]], sem0, add=True)
  pltpu.async_copy(ones_v, acc_sh.at[idx_d.at[1]], sem1, add=True)
  n_half = n_chunks // 2

  def step(k, carry):
    j0 = 2 * k
    pltpu.make_async_copy(ones_v, acc_sh.at[idx_d.at[j0]], sem0).wait()

    @pl.when(k < n_half - 1)
    def _():
      pltpu.async_copy(ones_v, acc_sh.at[idx_d.at[j0 + 2]], sem0, add=True)

    pltpu.make_async_copy(ones_v, acc_sh.at[idx_d.at[j0 + 1]], sem1).wait()

    @pl.when(k < n_half - 1)
    def _():
      pltpu.async_copy(ones_v, acc_sh.at[idx_d.at[j0 + 3]], sem1, add=True)

    return carry

  lax.fori_loop(0, n_half, step, 0)
  plsc.subcore_barrier()
  pltpu.sync_copy(acc_sh.at[pl.ds(s * rows_pt, rows_pt)],
                  out_hbm.at[c, pl.ds(s * rows_pt, rows_pt)])


def _sc_degree(dst_resh, ones_blk, zeros_w):
  n_nodes, width = zeros_w.shape
  n_chunks = dst_resh.shape[1]
  mesh = plsc.VectorSubcoreMesh(core_axis_name="c", subcore_axis_name="s")
  return pl.kernel(
      functools.partial(_deg_body, n_nodes, n_chunks, width),
      out_type=jax.ShapeDtypeStruct((NC, n_nodes, width), jnp.float32),
      mesh=mesh,
      scratch_types=[
          pltpu.VMEM((n_chunks, CHUNK), jnp.int32),
          pltpu.VMEM((CHUNK, width), jnp.float32),
          pltpu.VMEM_SHARED((n_nodes, width), jnp.float32),
          pltpu.SemaphoreType.DMA,
          pltpu.SemaphoreType.DMA,
      ],
  )(dst_resh, ones_blk, zeros_w)


def _pre_body(degp_ref, x_ref, w1t_ref, dinv_ref, g_ref, gh_ref):
  deg = degp_ref[0] + degp_ref[1]
  deg = deg[:, 0:1] + 1.0
  dinv = lax.rsqrt(deg)
  h = jnp.dot(x_ref[...], w1t_ref[...], preferred_element_type=jnp.float32)
  g = h * dinv
  dinv_ref[...] = dinv
  g_ref[...] = g
  gh_ref[...] = g * 0.5


def _tc_pre(degp, x, w1t):
  n, d = x.shape
  h = w1t.shape[1]
  return pl.pallas_call(
      _pre_body,
      out_shape=[
          jax.ShapeDtypeStruct((n, 1), jnp.float32),
          jax.ShapeDtypeStruct((n, h), jnp.float32),
          jax.ShapeDtypeStruct((n, h), jnp.float32),
      ],
  )(degp, x, w1t)


def _mid_body(n_nodes, p_ref, dinv_ref, b1_ref, w2t_ref, g_ref, gh_ref):
  h = p_ref[0] + p_ref[1]
  h = h * dinv_ref[...] + b1_ref[...]
  h = jnp.maximum(h, 0.0)
  g = jnp.dot(h, w2t_ref[...], preferred_element_type=jnp.float32)
  g = g * dinv_ref[...]
  row = lax.broadcasted_iota(jnp.int32, g.shape, 0)
  g = jnp.where(row < n_nodes, g, 0.0)
  g_ref[...] = g
  gh_ref[...] = g * 0.5


def _tc_mid(p, dinv, b1, w2t, n_nodes):
  n = p.shape[1]
  h = w2t.shape[1]
  return pl.pallas_call(
      functools.partial(_mid_body, n_nodes),
      out_shape=[
          jax.ShapeDtypeStruct((n, h), jnp.float32),
          jax.ShapeDtypeStruct((n, h), jnp.float32),
      ],
  )(p, dinv, b1, w2t)


def _post_body(p_ref, dinv_ref, b2_ref, wlt_ref, bl_ref, out_ref):
  h = p_ref[0] + p_ref[1]
  h = h * dinv_ref[...] + b2_ref[...]
  h = jnp.maximum(h, 0.0)
  out_ref[...] = (
      jnp.dot(h, wlt_ref[...], preferred_element_type=jnp.float32)
      + bl_ref[...])


def _tc_post(p, dinv, b2, wlt, bl):
  n = p.shape[1]
  n_out = wlt.shape[1]
  return pl.pallas_call(
      _post_body,
      out_shape=jax.ShapeDtypeStruct((n, n_out), jnp.float32),
  )(p, dinv, b2, wlt, bl)


def kernel(x, edge_index, W1, b1, W2, b2, Wl, bl):
  n_nodes, d = x.shape
  n_edges = edge_index.shape[1]
  n_workers = NC * NS

  n_pad = NS * 8 * ((n_nodes + NS * 8 - 1) // (NS * 8))
  x = jnp.pad(x, ((0, n_pad - n_nodes), (0, 0)))

  per_w = CHUNK * 8 * (
      (n_edges + n_workers * CHUNK * 8 - 1) // (n_workers * CHUNK * 8))
  e_pad = n_workers * per_w
  pad_n = e_pad - n_edges
  pad_i = jnp.arange(pad_n, dtype=jnp.int32)
  pad_src = n_nodes + pad_i % (n_pad - n_nodes)
  src_full = jnp.concatenate([edge_index[0], pad_src])
  dst_agg = jnp.concatenate([edge_index[1], pad_i % n_pad])
  dst_deg = jnp.concatenate([edge_index[1], pad_src])

  src32 = src_full.reshape(n_workers, per_w // CHUNK, CHUNK)
  dst32a = dst_agg.reshape(n_workers, per_w // CHUNK, CHUNK)
  dst32d = dst_deg.reshape(n_workers, per_w // CHUNK, CHUNK)

  ones_blk = jnp.ones((CHUNK, d), jnp.float32)
  zeros_w = jnp.zeros((n_pad, d), jnp.float32)
  degp = _sc_degree(dst32d, ones_blk, zeros_w)
  dinv, g1, gh1 = _tc_pre(degp, x, W1.T)
  p1 = _sc_aggregate(g1, gh1, src32, dst32a)
  g2, gh2 = _tc_mid(p1, dinv, b1.reshape(1, -1), W2.T, n_nodes)
  p2 = _sc_aggregate(g2, gh2, src32, dst32a)
  out = _tc_post(p2, dinv, b2.reshape(1, -1), Wl.T, bl.reshape(1, -1))
  return out[:n_nodes]
